# baseline (speedup 1.0000x reference)
import jax
import jax.numpy as jnp
from jax.experimental import pallas as pl
from jax.experimental.pallas import tpu as pltpu

NUM_TIMESTEPS = 1000
BETA_START = 0.0001
BETA_END = 0.02

_CH = 16
_D = 6


def _body(a_ref, c_ref, x_hbm, n_hbm, o_hbm, xb, nb, ob, xsem, nsem, osem):
    i = pl.program_id(0)
    nch = pl.num_programs(0)
    s = i % _D

    def in_copies(chunk, slot):
        cx = pltpu.make_async_copy(
            x_hbm.at[pl.ds(chunk * _CH, _CH)], xb.at[slot], xsem.at[slot])
        cn = pltpu.make_async_copy(
            n_hbm.at[pl.ds(chunk * _CH, _CH)], nb.at[slot], nsem.at[slot])
        return cx, cn

    def out_copy(chunk, slot):
        return pltpu.make_async_copy(
            ob.at[slot], o_hbm.at[pl.ds(chunk * _CH, _CH)], osem.at[slot])

    @pl.when(i == 0)
    def _():
        for d in range(_D):
            cx, cn = in_copies(d, d)
            cx.start()
            cn.start()

    cx, cn = in_copies(i, s)
    cx.wait()
    cn.wait()

    @pl.when(i >= _D)
    def _():
        out_copy(i - _D, s).wait()

    a = a_ref[pl.ds(i * _CH, _CH)].reshape(_CH, 1, 1)
    c = c_ref[pl.ds(i * _CH, _CH)].reshape(_CH, 1, 1)
    ob[s] = a * xb[s] + c * nb[s]
    out_copy(i, s).start()

    @pl.when(i + _D < nch)
    def _():
        cx, cn = in_copies(i + _D, s)
        cx.start()
        cn.start()

    @pl.when(i == nch - 1)
    def _():
        for d in range(_D):
            out_copy(i, d).wait()


def _tables():
    betas = jnp.linspace(BETA_START, BETA_END, NUM_TIMESTEPS, dtype=jnp.float32)
    alphas_cumprod = jnp.cumprod(1.0 - betas, axis=0)
    sac = jnp.sqrt(alphas_cumprod)
    somac = jnp.sqrt(1.0 - alphas_cumprod)
    return sac, somac


def kernel(x_start, t, noise):
    B = x_start.shape[0]
    F = x_start.size // B
    S = F // 128
    x = x_start.reshape(B, S, 128)
    n = noise.reshape(B, S, 128)
    sac, somac = _tables()
    t32 = t.astype(jnp.int32)
    a = jnp.take(sac, t32, axis=0).reshape(B, 1)
    c = jnp.take(somac, t32, axis=0).reshape(B, 1)

    out = pl.pallas_call(
        _body,
        grid=(B // _CH,),
        in_specs=[
            pl.BlockSpec((B, 1), lambda i: (0, 0)),
            pl.BlockSpec((B, 1), lambda i: (0, 0)),
            pl.BlockSpec(memory_space=pl.MemorySpace.ANY),
            pl.BlockSpec(memory_space=pl.MemorySpace.ANY),
        ],
        out_specs=pl.BlockSpec(memory_space=pl.MemorySpace.ANY),
        out_shape=jax.ShapeDtypeStruct((B, S, 128), jnp.float32),
        scratch_shapes=[
            pltpu.VMEM((_D, _CH, S, 128), jnp.float32),
            pltpu.VMEM((_D, _CH, S, 128), jnp.float32),
            pltpu.VMEM((_D, _CH, S, 128), jnp.float32),
            pltpu.SemaphoreType.DMA((_D,)),
            pltpu.SemaphoreType.DMA((_D,)),
            pltpu.SemaphoreType.DMA((_D,)),
        ],
    )(a, c, x, n)
    return out.reshape(x_start.shape)
